# baseline (speedup 1.0000x reference)
import functools

import jax
import jax.numpy as jnp
from jax import lax
from jax.experimental import pallas as pl
from jax.experimental.pallas import tpu as pltpu
from jax.experimental.pallas import tpu_sc as plsc

B = 64
N = 1024
D = 2048
L = 256
NC = 2
NS = 16
NW = NC * NS
DW = D // NW
G = 4

_mesh = plsc.VectorSubcoreMesh(
    core_axis_name="c", subcore_axis_name="s", num_cores=1, num_subcores=NS
)


@functools.partial(
    pl.kernel,
    out_type=jax.ShapeDtypeStruct((NS, B, 2, DW // 2), jnp.int16),
    mesh=_mesh,
    compiler_params=pltpu.CompilerParams(use_tc_tiling_on_sc=False),
    scratch_types=[
        pltpu.VMEM((G, N), jnp.float32),
        pltpu.VMEM((G, N), jnp.int32),
        pltpu.VMEM((N, 2, DW // 2), jnp.int16),
        pltpu.VMEM((L, 2, DW // 2), jnp.int16),
        pltpu.VMEM((B, 2, DW // 2), jnp.int16),
    ],
)
def _sc_encode(x_hbm, pos_hbm, val_hbm, out_hbm, xrow_v, idx_v, pos_v, val_v, out_v):
    wid = lax.axis_index("s")

    pltpu.sync_copy(pos_hbm.at[wid], pos_v)
    pltpu.sync_copy(val_hbm.at[wid], val_v)

    def group_body(bg, carry):
        pltpu.sync_copy(x_hbm.at[pl.ds(bg * G, G)], xrow_v)

        def qg(g, carry2):
            for k in range(G):
                v = xrow_v[k, pl.ds(g * 16, 16)] * 255.0
                t = v.astype(jnp.int32)
                f = v - t.astype(jnp.float32)
                up = (f > 0.5) | ((f == 0.5) & ((t & 1) == 1))
                r = jnp.where(up, t + 1, t)
                r = jnp.minimum(jnp.maximum(r, 0), 255)
                idx_v[k, pl.ds(g * 16, 16)] = r
            return carry2

        lax.fori_loop(0, N // 16, qg, 0)

        def gbody(g, accs):
            accs = list(accs)
            ivs = [idx_v[k, pl.ds(g * 16, 16)] for k in range(G)]
            base = g * 16
            for j in range(16):
                n = base + j
                p0 = pos_v[n, 0, pl.ds(0, 32)]
                p1 = pos_v[n, 1, pl.ds(0, 32)]
                for k in range(G):
                    r = ivs[k][j]
                    accs[2 * k] = accs[2 * k] + p0 * val_v[r, 0, pl.ds(0, 32)]
                    accs[2 * k + 1] = accs[2 * k + 1] + p1 * val_v[r, 1, pl.ds(0, 32)]
            return tuple(accs)

        z = jnp.zeros((32,), jnp.int16)
        accs = lax.fori_loop(0, N // 16, gbody, (z,) * (2 * G))
        one = jnp.int16(1)
        mone = jnp.int16(-1)
        for k in range(G):
            s0 = jnp.maximum(jnp.minimum(accs[2 * k] - one, one), mone)
            s1 = jnp.maximum(jnp.minimum(accs[2 * k + 1] - one, one), mone)
            out_v[bg * G + k, 0, pl.ds(0, 32)] = s0
            out_v[bg * G + k, 1, pl.ds(0, 32)] = s1
        return carry

    lax.fori_loop(0, B // G, group_body, 0)
    pltpu.sync_copy(out_v, out_hbm.at[wid])


def kernel(x, position_weight, value_weight):
    xf = x.reshape(B, N)
    pos16 = (
        position_weight.astype(jnp.int16)
        .reshape(N, NW, 2, DW // 2)
        .transpose(1, 0, 2, 3)
    )
    val16 = (
        value_weight.astype(jnp.int16)
        .reshape(L, NW, 2, DW // 2)
        .transpose(1, 0, 2, 3)
    )
    out_a = _sc_encode(xf, pos16[:NS], val16[:NS])
    out_b = _sc_encode(xf, pos16[NS:], val16[NS:])
    out_t = jnp.concatenate([out_a, out_b], axis=0)
    return (
        out_t.reshape(NW, B, DW).transpose(1, 0, 2).reshape(B, D).astype(jnp.float32)
    )


# speedup vs baseline: 1.1546x; 1.1546x over previous
import functools

import jax
import jax.numpy as jnp
from jax import lax
from jax.experimental import pallas as pl
from jax.experimental.pallas import tpu as pltpu
from jax.experimental.pallas import tpu_sc as plsc

B = 64
N = 1024
D = 2048
L = 256
NC = 2
NS = 16
NW = NC * NS
DW = D // NW
G = 4

_mesh = plsc.VectorSubcoreMesh(
    core_axis_name="c", subcore_axis_name="s", num_cores=NC, num_subcores=NS
)


@functools.partial(
    pl.kernel,
    out_type=jax.ShapeDtypeStruct((NW, B, 4, DW // 4), jnp.int32),
    mesh=_mesh,
    compiler_params=pltpu.CompilerParams(
        use_tc_tiling_on_sc=False, needs_layout_passes=False
    ),
    scratch_types=[
        pltpu.VMEM((G, N), jnp.float32),
        pltpu.VMEM((G, N), jnp.int32),
        pltpu.VMEM((N, 1, DW), jnp.int8),
        pltpu.VMEM((L, 1, DW), jnp.int8),
        pltpu.VMEM((B, 4, DW // 4), jnp.int32),
    ],
)
def _sc_encode(x_hbm, pos_hbm, val_hbm, out_hbm, xrow_v, idx_v, pos_v, val_v, out_v):
    c = lax.axis_index("c")
    s = lax.axis_index("s")
    wid = s * NC + c

    pltpu.sync_copy(pos_hbm.at[wid], pos_v)
    pltpu.sync_copy(val_hbm.at[wid], val_v)

    def group_body(bg, carry):
        pltpu.sync_copy(x_hbm.at[pl.ds(bg * G, G)], xrow_v)

        def qg(g, carry2):
            for k in range(G):
                v = xrow_v[k, pl.ds(g * 16, 16)] * 255.0
                t = v.astype(jnp.int32)
                f = v - t.astype(jnp.float32)
                up = (f > 0.5) | ((f == 0.5) & ((t & 1) == 1))
                r = jnp.where(up, t + 1, t)
                r = jnp.minimum(jnp.maximum(r, 0), 255)
                idx_v[k, pl.ds(g * 16, 16)] = r
            return carry2

        lax.fori_loop(0, N // 16, qg, 0)

        def gbody(g, accs):
            accs = list(accs)
            ivs = [idx_v[k, pl.ds(g * 16, 16)] for k in range(G)]
            base = g * 16
            z8 = jnp.zeros((DW,), jnp.int8)
            a8 = [z8] * G
            for j in range(16):
                n = base + j
                p = pos_v[n, 0, pl.ds(0, DW)]
                for k in range(G):
                    r = ivs[k][j]
                    a8[k] = a8[k] + p * val_v[r, 0, pl.ds(0, DW)]
            c24 = jnp.int32(24)
            c16 = jnp.int32(16)
            c8 = jnp.int32(8)
            for k in range(G):
                w = plsc.bitcast(a8[k], jnp.int32)
                b0 = lax.shift_right_arithmetic(lax.shift_left(w, c24), c24)
                b1 = lax.shift_right_arithmetic(lax.shift_left(w, c16), c24)
                b2 = lax.shift_right_arithmetic(lax.shift_left(w, c8), c24)
                b3 = lax.shift_right_arithmetic(w, c24)
                accs[4 * k] = accs[4 * k] + b0
                accs[4 * k + 1] = accs[4 * k + 1] + b1
                accs[4 * k + 2] = accs[4 * k + 2] + b2
                accs[4 * k + 3] = accs[4 * k + 3] + b3
            return tuple(accs)

        z = jnp.zeros((16,), jnp.int32)
        accs = lax.fori_loop(0, N // 16, gbody, (z,) * (4 * G))
        one = jnp.int32(1)
        mone = jnp.int32(-1)
        for k in range(G):
            for q in range(4):
                sq = jnp.maximum(jnp.minimum(accs[4 * k + q] - one, one), mone)
                out_v[bg * G + k, q, pl.ds(0, 16)] = sq
        return carry

    lax.fori_loop(0, B // G, group_body, 0)
    pltpu.sync_copy(out_v, out_hbm.at[wid])


def kernel(x, position_weight, value_weight):
    xf = x.reshape(B, N)
    perm = jnp.arange(D).reshape(NW, 4, 16).transpose(0, 2, 1).reshape(D)
    pos8 = (
        position_weight[:, perm]
        .astype(jnp.int8)
        .reshape(N, NW, 1, DW)
        .transpose(1, 0, 2, 3)
    )
    val8 = (
        value_weight[:, perm]
        .astype(jnp.int8)
        .reshape(L, NW, 1, DW)
        .transpose(1, 0, 2, 3)
    )
    out_t = _sc_encode(xf, pos8, val8)
    return (
        out_t.reshape(NW, B, DW).transpose(1, 0, 2).reshape(B, D).astype(jnp.float32)
    )


# speedup vs baseline: 1.7218x; 1.4913x over previous
import functools

import jax
import jax.numpy as jnp
from jax import lax
from jax.experimental import pallas as pl
from jax.experimental.pallas import tpu as pltpu
from jax.experimental.pallas import tpu_sc as plsc

B = 64
N = 1024
D = 2048
L = 256
NC = 2
NS = 16
NW = NC * NS
DW = D // NW
DWW = DW // 4
G = 4

_mesh = plsc.VectorSubcoreMesh(
    core_axis_name="c", subcore_axis_name="s", num_cores=NC, num_subcores=NS
)


@functools.partial(
    pl.kernel,
    out_type=jax.ShapeDtypeStruct((NW, B, 4, DW // 4), jnp.int32),
    mesh=_mesh,
    compiler_params=pltpu.CompilerParams(use_tc_tiling_on_sc=False),
    scratch_types=[
        pltpu.VMEM((G, N), jnp.float32),
        pltpu.VMEM((G, N), jnp.int32),
        pltpu.VMEM((N, 1, DWW), jnp.int32),
        pltpu.VMEM((L, 1, DWW), jnp.int32),
        pltpu.VMEM((B, 4, DW // 4), jnp.int32),
    ],
)
def _sc_encode(x_hbm, pos_hbm, val_hbm, out_hbm, xrow_v, idx_v, pos_v, val_v, out_v):
    c = lax.axis_index("c")
    s = lax.axis_index("s")
    wid = s * NC + c

    pltpu.sync_copy(pos_hbm.at[wid], pos_v)
    pltpu.sync_copy(val_hbm.at[wid], val_v)

    def group_body(bg, carry):
        pltpu.sync_copy(x_hbm.at[pl.ds(bg * G, G)], xrow_v)

        def qg(g, carry2):
            for k in range(G):
                v = xrow_v[k, pl.ds(g * 16, 16)] * 255.0
                t = v.astype(jnp.int32)
                f = v - t.astype(jnp.float32)
                up = (f > 0.5) | ((f == 0.5) & ((t & 1) == 1))
                r = jnp.where(up, t + 1, t)
                r = jnp.minimum(jnp.maximum(r, 0), 255)
                idx_v[k, pl.ds(g * 16, 16)] = r
            return carry2

        lax.fori_loop(0, N // 16, qg, 0)

        def gbody(g, accs):
            accs = list(accs)
            ivs = [idx_v[k, pl.ds(g * 16, 16)] for k in range(G)]
            base = g * 16
            zz = jnp.zeros((DWW,), jnp.int32)
            a = [zz] * G
            for j in range(16):
                n = base + j
                p = pos_v[n, 0, pl.ds(0, DWW)]
                for k in range(G):
                    r = ivs[k][j]
                    a[k] = a[k] + (p ^ val_v[r, 0, pl.ds(0, DWW)])
            mff = jnp.int32(0xFF)
            for k in range(G):
                w = a[k]
                accs[4 * k] = accs[4 * k] + (w & mff)
                accs[4 * k + 1] = accs[4 * k + 1] + (
                    lax.shift_right_logical(w, 8) & mff
                )
                accs[4 * k + 2] = accs[4 * k + 2] + (
                    lax.shift_right_logical(w, 16) & mff
                )
                accs[4 * k + 3] = accs[4 * k + 3] + lax.shift_right_logical(w, 24)
            return tuple(accs)

        z = jnp.zeros((16,), jnp.int32)
        accs = lax.fori_loop(0, N // 16, gbody, (z,) * (4 * G))
        one = jnp.int32(1)
        mone = jnp.int32(-1)
        big = jnp.int32(1023)
        for k in range(G):
            for q in range(4):
                cnt = accs[4 * k + q]
                sq = jnp.maximum(jnp.minimum(big - cnt - cnt, one), mone)
                out_v[bg * G + k, q, pl.ds(0, 16)] = sq
        return carry

    lax.fori_loop(0, B // G, group_body, 0)
    pltpu.sync_copy(out_v, out_hbm.at[wid])


def kernel(x, position_weight, value_weight):
    xf = x.reshape(B, N)
    perm = jnp.arange(D).reshape(NW, 4, 16).transpose(0, 2, 1).reshape(D)
    pos01 = (position_weight[:, perm] < 0).astype(jnp.int8)
    val01 = (value_weight[:, perm] < 0).astype(jnp.int8)
    pos32 = lax.bitcast_convert_type(pos01.reshape(N, D // 4, 4), jnp.int32)
    val32 = lax.bitcast_convert_type(val01.reshape(L, D // 4, 4), jnp.int32)
    pos32 = pos32.reshape(N, NW, 1, DWW).transpose(1, 0, 2, 3)
    val32 = val32.reshape(L, NW, 1, DWW).transpose(1, 0, 2, 3)
    out_t = _sc_encode(xf, pos32, val32)
    return (
        out_t.reshape(NW, B, DW).transpose(1, 0, 2).reshape(B, D).astype(jnp.float32)
    )


# speedup vs baseline: 1.8361x; 1.0664x over previous
import functools

import jax
import jax.numpy as jnp
from jax import lax
from jax.experimental import pallas as pl
from jax.experimental.pallas import tpu as pltpu
from jax.experimental.pallas import tpu_sc as plsc

B = 64
N = 1024
D = 2048
L = 256
NC = 2
NS = 16
NW = NC * NS
DW = D // NW
DWW = DW // 4
G = 4

_mesh = plsc.VectorSubcoreMesh(
    core_axis_name="c", subcore_axis_name="s", num_cores=NC, num_subcores=NS
)


@functools.partial(
    pl.kernel,
    out_type=jax.ShapeDtypeStruct((NW, B, 4, DW // 4), jnp.int32),
    mesh=_mesh,
    compiler_params=pltpu.CompilerParams(use_tc_tiling_on_sc=False),
    scratch_types=[
        pltpu.VMEM((G, N), jnp.float32),
        pltpu.VMEM((G, N), jnp.int32),
        pltpu.VMEM((N, 1, DWW), jnp.int32),
        pltpu.VMEM((L, 1, DWW), jnp.int32),
        pltpu.VMEM((B, 4, DW // 4), jnp.int32),
    ],
)
def _sc_encode(x_hbm, pos_hbm, val_hbm, out_hbm, xrow_v, idx_v, pos_v, val_v, out_v):
    c = lax.axis_index("c")
    s = lax.axis_index("s")
    wid = s * NC + c

    pltpu.sync_copy(pos_hbm.at[wid], pos_v)
    pltpu.sync_copy(val_hbm.at[wid], val_v)

    def group_body(bg, carry):
        pltpu.sync_copy(x_hbm.at[pl.ds(bg * G, G)], xrow_v)

        def qg(g, carry2):
            for k in range(G):
                v = xrow_v[k, pl.ds(g * 16, 16)] * 255.0
                t = v.astype(jnp.int32)
                f = v - t.astype(jnp.float32)
                up = (f > 0.5) | ((f == 0.5) & ((t & 1) == 1))
                r = jnp.where(up, t + 1, t)
                r = jnp.minimum(jnp.maximum(r, 0), 255)
                idx_v[k, pl.ds(g * 16, 16)] = r
            return carry2

        lax.fori_loop(0, N // 16, qg, 0)

        def gbody(g, accs):
            accs = list(accs)
            ivs = [idx_v[k, pl.ds(g * 16, 16)] for k in range(G)]
            base = g * 16
            zz = jnp.zeros((DWW,), jnp.int32)
            a = [zz] * G
            for j in range(16):
                n = base + j
                p = pos_v[n, 0, pl.ds(0, DWW)]
                for k in range(G):
                    r = ivs[k][j]
                    a[k] = a[k] + (p ^ val_v[r, 0, pl.ds(0, DWW)])
            mff = jnp.int32(0xFF)
            for k in range(G):
                w = a[k]
                accs[4 * k] = accs[4 * k] + (w & mff)
                accs[4 * k + 1] = accs[4 * k + 1] + (
                    lax.shift_right_logical(w, 8) & mff
                )
                accs[4 * k + 2] = accs[4 * k + 2] + (
                    lax.shift_right_logical(w, 16) & mff
                )
                accs[4 * k + 3] = accs[4 * k + 3] + lax.shift_right_logical(w, 24)
            return tuple(accs)

        z = jnp.zeros((16,), jnp.int32)
        accs = lax.fori_loop(0, N // 16, gbody, (z,) * (4 * G))
        one = jnp.int32(1)
        mone = jnp.int32(-1)
        big = jnp.int32(1023)
        for k in range(G):
            for q in range(4):
                cnt = accs[4 * k + q]
                sq = jnp.maximum(jnp.minimum(big - cnt - cnt, one), mone)
                out_v[bg * G + k, q, pl.ds(0, 16)] = sq
        return carry

    lax.fori_loop(0, B // G, group_body, 0)
    pltpu.sync_copy(out_v, out_hbm.at[wid])


def kernel(x, position_weight, value_weight):
    xf = x.reshape(B, N)
    pos01 = (position_weight < 0).astype(jnp.int8)
    val01 = (value_weight < 0).astype(jnp.int8)
    pos32 = lax.bitcast_convert_type(pos01.reshape(N, D // 4, 4), jnp.int32)
    val32 = lax.bitcast_convert_type(val01.reshape(L, D // 4, 4), jnp.int32)
    pos32 = pos32.reshape(N, NW, 1, DWW).transpose(1, 0, 2, 3)
    val32 = val32.reshape(L, NW, 1, DWW).transpose(1, 0, 2, 3)
    out_t = _sc_encode(xf, pos32, val32)
    return (
        out_t.transpose(1, 0, 3, 2).reshape(B, D).astype(jnp.float32)
    )


# speedup vs baseline: 2.7288x; 1.4861x over previous
import functools

import jax
import jax.numpy as jnp
from jax import lax
from jax.experimental import pallas as pl
from jax.experimental.pallas import tpu as pltpu
from jax.experimental.pallas import tpu_sc as plsc

B = 64
N = 1024
D = 2048
L = 256
NC = 2
NS = 16
NW = NC * NS
DW = D // NW
DWW = DW // 4
G = 4

_mesh = plsc.VectorSubcoreMesh(
    core_axis_name="c", subcore_axis_name="s", num_cores=NC, num_subcores=NS
)


@functools.partial(
    pl.kernel,
    out_type=jax.ShapeDtypeStruct((NW, B, 4, DW // 4), jnp.float32),
    mesh=_mesh,
    compiler_params=pltpu.CompilerParams(use_tc_tiling_on_sc=False),
    scratch_types=[
        pltpu.VMEM((G, N), jnp.float32),
        pltpu.VMEM((G, N), jnp.int32),
        pltpu.VMEM((N, 1, DWW), jnp.int32),
        pltpu.VMEM((L, 1, DWW), jnp.int32),
        pltpu.VMEM((B, 4, DW // 4), jnp.float32),
    ],
)
def _sc_encode(x_hbm, pos_hbm, val_hbm, out_hbm, xrow_v, idx_v, pos_v, val_v, out_v):
    c = lax.axis_index("c")
    s = lax.axis_index("s")
    wid = s * NC + c

    pltpu.sync_copy(pos_hbm.at[:, pl.ds(wid, 1), :], pos_v)
    pltpu.sync_copy(val_hbm.at[:, pl.ds(wid, 1), :], val_v)

    def group_body(bg, carry):
        pltpu.sync_copy(x_hbm.at[pl.ds(bg * G, G)], xrow_v)

        def qg(g, carry2):
            for k in range(G):
                v = xrow_v[k, pl.ds(g * 16, 16)] * 255.0
                t = v.astype(jnp.int32)
                f = v - t.astype(jnp.float32)
                up = (f > 0.5) | ((f == 0.5) & ((t & 1) == 1))
                r = jnp.where(up, t + 1, t)
                r = jnp.minimum(jnp.maximum(r, 0), 255)
                idx_v[k, pl.ds(g * 16, 16)] = r
            return carry2

        lax.fori_loop(0, N // 16, qg, 0)

        def gbody(g, accs):
            accs = list(accs)
            ivs = [idx_v[k, pl.ds(g * 16, 16)] for k in range(G)]
            base = g * 16
            zz = jnp.zeros((DWW,), jnp.int32)
            a = [zz] * G
            for j in range(16):
                n = base + j
                p = pos_v[n, 0, pl.ds(0, DWW)]
                for k in range(G):
                    r = ivs[k][j]
                    a[k] = a[k] + (p ^ val_v[r, 0, pl.ds(0, DWW)])
            mff = jnp.int32(0xFF)
            for k in range(G):
                w = a[k]
                accs[4 * k] = accs[4 * k] + (w & mff)
                accs[4 * k + 1] = accs[4 * k + 1] + (
                    lax.shift_right_logical(w, 8) & mff
                )
                accs[4 * k + 2] = accs[4 * k + 2] + (
                    lax.shift_right_logical(w, 16) & mff
                )
                accs[4 * k + 3] = accs[4 * k + 3] + lax.shift_right_logical(w, 24)
            return tuple(accs)

        z = jnp.zeros((16,), jnp.int32)
        accs = lax.fori_loop(0, N // 16, gbody, (z,) * (4 * G))
        big = jnp.int32(1023)
        for k in range(G):
            for q in range(4):
                cnt = accs[4 * k + q]
                sq = (big - cnt - cnt).astype(jnp.float32)
                sq = jnp.maximum(jnp.minimum(sq, 1.0), -1.0)
                out_v[bg * G + k, q, pl.ds(0, 16)] = sq
        return carry

    lax.fori_loop(0, B // G, group_body, 0)
    pltpu.sync_copy(out_v, out_hbm.at[wid])


def kernel(x, position_weight, value_weight):
    xf = x.reshape(B, N)
    pos01 = (position_weight < 0).astype(jnp.int8)
    val01 = (value_weight < 0).astype(jnp.int8)
    pos32 = lax.bitcast_convert_type(pos01.reshape(N, D // 4, 4), jnp.int32)
    val32 = lax.bitcast_convert_type(val01.reshape(L, D // 4, 4), jnp.int32)
    pos32 = pos32.reshape(N, NW, DWW)
    val32 = val32.reshape(L, NW, DWW)
    out_t = _sc_encode(xf, pos32, val32)
    return out_t.transpose(1, 0, 3, 2).reshape(B, D)


# speedup vs baseline: 2.8060x; 1.0283x over previous
import functools

import jax
import jax.numpy as jnp
from jax import lax
from jax.experimental import pallas as pl
from jax.experimental.pallas import tpu as pltpu
from jax.experimental.pallas import tpu_sc as plsc

B = 64
N = 1024
D = 2048
L = 256
NC = 2
NS = 16
NW = NC * NS
DW = D // NW
DWW = DW // 4
G = 4

_mesh = plsc.VectorSubcoreMesh(
    core_axis_name="c", subcore_axis_name="s", num_cores=NC, num_subcores=NS
)


@functools.partial(
    pl.kernel,
    out_type=jax.ShapeDtypeStruct((NW, B, 4, DW // 4), jnp.float32),
    mesh=_mesh,
    compiler_params=pltpu.CompilerParams(use_tc_tiling_on_sc=False),
    scratch_types=[
        pltpu.VMEM((G, N), jnp.float32),
        pltpu.VMEM((G, N), jnp.int32),
        pltpu.VMEM((N, 1, DWW), jnp.int32),
        pltpu.VMEM((L, 1, DWW), jnp.int32),
        pltpu.VMEM((B, 4, DW // 4), jnp.float32),
    ],
)
def _sc_encode(x_hbm, pos_hbm, val_hbm, out_hbm, xrow_v, idx_v, pos_v, val_v, out_v):
    c = lax.axis_index("c")
    s = lax.axis_index("s")
    wid = s * NC + c

    pltpu.sync_copy(pos_hbm.at[:, pl.ds(wid, 1), :], pos_v)
    pltpu.sync_copy(val_hbm.at[:, pl.ds(wid, 1), :], val_v)

    def group_body(bg, carry):
        pltpu.sync_copy(x_hbm.at[pl.ds(bg * G, G)], xrow_v)

        def qg(g, carry2):
            magic = jnp.float32(12582912.0)
            for k in range(G):
                v = xrow_v[k, pl.ds(g * 16, 16)] * 255.0
                r = ((v + magic) - magic).astype(jnp.int32)
                r = jnp.minimum(jnp.maximum(r, 0), 255)
                idx_v[k, pl.ds(g * 16, 16)] = r
            return carry2

        lax.fori_loop(0, N // 16, qg, 0)

        def gbody(g, accs):
            accs = list(accs)
            ivs = [idx_v[k, pl.ds(g * 16, 16)] for k in range(G)]
            base = g * 16
            zz = jnp.zeros((DWW,), jnp.int32)
            a = [zz] * G
            for j in range(16):
                n = base + j
                p = pos_v[n, 0, pl.ds(0, DWW)]
                for k in range(G):
                    r = ivs[k][j]
                    a[k] = a[k] + (p ^ val_v[r, 0, pl.ds(0, DWW)])
            mff = jnp.int32(0xFF)
            for k in range(G):
                w = a[k]
                accs[4 * k] = accs[4 * k] + (w & mff)
                accs[4 * k + 1] = accs[4 * k + 1] + (
                    lax.shift_right_logical(w, 8) & mff
                )
                accs[4 * k + 2] = accs[4 * k + 2] + (
                    lax.shift_right_logical(w, 16) & mff
                )
                accs[4 * k + 3] = accs[4 * k + 3] + lax.shift_right_logical(w, 24)
            return tuple(accs)

        z = jnp.zeros((16,), jnp.int32)
        accs = lax.fori_loop(0, N // 16, gbody, (z,) * (4 * G))
        big = jnp.int32(1023)
        for k in range(G):
            for q in range(4):
                cnt = accs[4 * k + q]
                sq = (big - cnt - cnt).astype(jnp.float32)
                sq = jnp.maximum(jnp.minimum(sq, 1.0), -1.0)
                out_v[bg * G + k, q, pl.ds(0, 16)] = sq
        return carry

    lax.fori_loop(0, B // G, group_body, 0)
    pltpu.sync_copy(out_v, out_hbm.at[wid])


def kernel(x, position_weight, value_weight):
    xf = x.reshape(B, N)
    pos01 = (position_weight < 0).astype(jnp.int8)
    val01 = (value_weight < 0).astype(jnp.int8)
    pos32 = lax.bitcast_convert_type(pos01.reshape(N, D // 4, 4), jnp.int32)
    val32 = lax.bitcast_convert_type(val01.reshape(L, D // 4, 4), jnp.int32)
    pos32 = pos32.reshape(N, NW, DWW)
    val32 = val32.reshape(L, NW, DWW)
    out_t = _sc_encode(xf, pos32, val32)
    return out_t.transpose(1, 0, 3, 2).reshape(B, D)
